# baseline (speedup 1.0000x reference)
import jax
import jax.numpy as jnp
from jax.experimental import pallas as pl
from jax.experimental.pallas import tpu as pltpu

N = 4096
D = 512
K = 31
BM = 256
BISECT_ITERS = 16
N_STRIP = 5


def _fused_body(x_ref, w0_ref, w1_ref, out_ref, emb_ref, buf_ref):
    i = pl.program_id(0)
    num = pl.num_programs(0)

    def matmul_into(blk, slot):
        rows = emb_ref[pl.ds(blk * BM, BM), :]
        buf_ref[slot] = jax.lax.dot_general(
            rows, emb_ref[:],
            dimension_numbers=(((1,), (1,)), ((), ())),
            preferred_element_type=jnp.float32,
        )

    @pl.when(i == 0)
    def _encode():
        h = x_ref[:] * w0_ref[:]
        h = jnp.maximum(h, 0.0)
        h = h * w1_ref[:]
        s = jnp.sum(h * h, axis=-1, keepdims=True)
        n = jnp.sqrt(s)
        emb_ref[:] = h / jnp.maximum(n, 1e-12)
        matmul_into(0, 0)

    @pl.when(i < num - 1)
    def _next():
        matmul_into(i + 1, (i + 1) % 2)

    a = buf_ref[i % 2]

    def body(_, carry):
        lo, hi, clo = carry
        mid = (lo + hi) * 0.5
        cnt = jnp.sum(jnp.where(a >= mid, 1.0, 0.0), axis=1, keepdims=True)
        ge = cnt >= K
        return (jnp.where(ge, mid, lo),
                jnp.where(ge, hi, mid),
                jnp.where(ge, cnt, clo))

    lo0 = jnp.full((BM, 1), -1.01, jnp.float32)
    hi0 = jnp.full((BM, 1), 1.01, jnp.float32)
    clo0 = jnp.full((BM, 1), float(N), jnp.float32)
    lo, _, clo = jax.lax.fori_loop(0, BISECT_ITERS, body, (lo0, hi0, clo0))
    candf = jnp.where(a >= lo, a, 2.0)
    t = jnp.min(candf, axis=1, keepdims=True)
    excess = clo - K
    for _ in range(N_STRIP):
        nxt = jnp.min(jnp.where(candf > t, candf, 2.0), axis=1, keepdims=True)
        t = jnp.where(excess >= 1.0, nxt, t)
        excess = excess - 1.0
    t = jnp.where(clo > K + N_STRIP, lo, t)
    t = jnp.maximum(t, 1.18e-38)
    out_ref[:] = jnp.where(a >= t, a, 0.0)


@jax.jit
def kernel(x, w0, w1):
    return pl.pallas_call(
        _fused_body,
        grid=(N // BM,),
        in_specs=[
            pl.BlockSpec((N, D), lambda i: (0, 0)),
            pl.BlockSpec((1, D), lambda i: (0, 0)),
            pl.BlockSpec((1, D), lambda i: (0, 0)),
        ],
        out_specs=pl.BlockSpec((BM, N), lambda i: (i, 0)),
        out_shape=jax.ShapeDtypeStruct((N, N), jnp.float32),
        scratch_shapes=[pltpu.VMEM((N, D), jnp.float32),
                        pltpu.VMEM((2, BM, N), jnp.float32)],
    )(x, w0.reshape(1, D), w1.reshape(1, D))


# speedup vs baseline: 1.0871x; 1.0871x over previous
import jax
import jax.numpy as jnp
from jax.experimental import pallas as pl
from jax.experimental.pallas import tpu as pltpu

N = 4096
D = 512
K = 31
BM = 512
BISECT_ITERS = 16
N_STRIP = 5


def _fused_body(x_ref, w0_ref, w1_ref, out_ref, emb_ref):
    i = pl.program_id(0)

    @pl.when(i == 0)
    def _encode():
        h = x_ref[:] * w0_ref[:]
        h = jnp.maximum(h, 0.0)
        h = h * w1_ref[:]
        s = jnp.sum(h * h, axis=-1, keepdims=True)
        n = jnp.sqrt(s)
        emb_ref[:] = h / jnp.maximum(n, 1e-12)

    rows = emb_ref[pl.ds(i * BM, BM), :]
    a = jax.lax.dot_general(
        rows, emb_ref[:],
        dimension_numbers=(((1,), (1,)), ((), ())),
        preferred_element_type=jnp.float32,
    )

    def body(_, carry):
        lo, hi, clo = carry
        mid = (lo + hi) * 0.5
        cnt = jnp.sum(jnp.where(a >= mid, 1.0, 0.0), axis=1, keepdims=True)
        ge = cnt >= K
        return (jnp.where(ge, mid, lo),
                jnp.where(ge, hi, mid),
                jnp.where(ge, cnt, clo))

    lo0 = jnp.full((BM, 1), -1.01, jnp.float32)
    hi0 = jnp.full((BM, 1), 1.01, jnp.float32)
    clo0 = jnp.full((BM, 1), float(N), jnp.float32)
    lo, _, clo = jax.lax.fori_loop(0, BISECT_ITERS, body, (lo0, hi0, clo0))
    candf = jnp.where(a >= lo, a, 2.0)
    t = jnp.min(candf, axis=1, keepdims=True)
    excess = clo - K
    for _ in range(N_STRIP):
        nxt = jnp.min(jnp.where(candf > t, candf, 2.0), axis=1, keepdims=True)
        t = jnp.where(excess >= 1.0, nxt, t)
        excess = excess - 1.0
    t = jnp.where(clo > K + N_STRIP, lo, t)
    t = jnp.maximum(t, 1.18e-38)
    out_ref[:] = jnp.where(a >= t, a, 0.0)


@jax.jit
def kernel(x, w0, w1):
    return pl.pallas_call(
        _fused_body,
        grid=(N // BM,),
        in_specs=[
            pl.BlockSpec((N, D), lambda i: (0, 0)),
            pl.BlockSpec((1, D), lambda i: (0, 0)),
            pl.BlockSpec((1, D), lambda i: (0, 0)),
        ],
        out_specs=pl.BlockSpec((BM, N), lambda i: (i, 0)),
        out_shape=jax.ShapeDtypeStruct((N, N), jnp.float32),
        scratch_shapes=[pltpu.VMEM((N, D), jnp.float32)],
    )(x, w0.reshape(1, D), w1.reshape(1, D))


# speedup vs baseline: 1.1550x; 1.0625x over previous
import jax
import jax.numpy as jnp
from jax.experimental import pallas as pl
from jax.experimental.pallas import tpu as pltpu

N = 4096
D = 512
K = 31
BM = 512
BISECT_ITERS = 14
N_STRIP = 4


def _encode_body(x_ref, w0_ref, w1_ref, emb_ref):
    h = x_ref[:] * w0_ref[:]
    h = jnp.maximum(h, 0.0)
    h = h * w1_ref[:]
    s = jnp.sum(h * h, axis=-1, keepdims=True)
    n = jnp.sqrt(s)
    emb_ref[:] = h / jnp.maximum(n, 1e-12)


def _main_body(emb_ref, out_ref, buf_ref):
    i = pl.program_id(0)
    num = pl.num_programs(0)

    def matmul_into(blk, slot):
        rows = emb_ref[pl.ds(blk * BM, BM), :]
        buf_ref[slot] = jax.lax.dot_general(
            rows, emb_ref[:],
            dimension_numbers=(((1,), (1,)), ((), ())),
            preferred_element_type=jnp.float32,
        )

    @pl.when(i == 0)
    def _prime():
        matmul_into(0, 0)

    @pl.when(i < num - 1)
    def _next():
        matmul_into(i + 1, (i + 1) % 2)

    a = buf_ref[i % 2]

    def body(_, carry):
        lo, hi, clo = carry
        mid = (lo + hi) * 0.5
        cnt = jnp.sum(jnp.where(a >= mid, 1.0, 0.0), axis=1, keepdims=True)
        ge = cnt >= K
        return (jnp.where(ge, mid, lo),
                jnp.where(ge, hi, mid),
                jnp.where(ge, cnt, clo))

    lo0 = jnp.full((BM, 1), -1.01, jnp.float32)
    hi0 = jnp.full((BM, 1), 1.01, jnp.float32)
    clo0 = jnp.full((BM, 1), float(N), jnp.float32)
    lo, _, clo = jax.lax.fori_loop(0, BISECT_ITERS, body, (lo0, hi0, clo0))
    candf = jnp.where(a >= lo, a, 2.0)
    t = jnp.min(candf, axis=1, keepdims=True)
    excess = clo - K
    for _ in range(N_STRIP):
        nxt = jnp.min(jnp.where(candf > t, candf, 2.0), axis=1, keepdims=True)
        t = jnp.where(excess >= 1.0, nxt, t)
        excess = excess - 1.0
    t = jnp.where(clo > K + N_STRIP, lo, t)
    t = jnp.maximum(t, 1.18e-38)
    out_ref[:] = jnp.where(a >= t, a, 0.0)


@jax.jit
def kernel(x, w0, w1):
    emb = pl.pallas_call(
        _encode_body,
        in_specs=[
            pl.BlockSpec((N, D), lambda: (0, 0)),
            pl.BlockSpec((1, D), lambda: (0, 0)),
            pl.BlockSpec((1, D), lambda: (0, 0)),
        ],
        out_specs=pl.BlockSpec((N, D), lambda: (0, 0)),
        out_shape=jax.ShapeDtypeStruct((N, D), jnp.float32),
    )(x, w0.reshape(1, D), w1.reshape(1, D))
    return pl.pallas_call(
        _main_body,
        grid=(N // BM,),
        in_specs=[pl.BlockSpec((N, D), lambda i: (0, 0))],
        out_specs=pl.BlockSpec((BM, N), lambda i: (i, 0)),
        out_shape=jax.ShapeDtypeStruct((N, N), jnp.float32),
        scratch_shapes=[pltpu.VMEM((2, BM, N), jnp.float32)],
    )(emb)


# speedup vs baseline: 1.2243x; 1.0600x over previous
import jax
import jax.numpy as jnp
from jax.experimental import pallas as pl
from jax.experimental.pallas import tpu as pltpu

N = 4096
D = 512
K = 31
BM = 512
BISECT_ITERS = 14
N_STRIP = 4


def _fused_body(x_ref, w0_ref, w1_ref, out_ref, emb_ref):
    i = pl.program_id(0)

    @pl.when(i == 0)
    def _encode():
        h = x_ref[:] * w0_ref[:]
        h = jnp.maximum(h, 0.0)
        h = h * w1_ref[:]
        s = jnp.sum(h * h, axis=-1, keepdims=True)
        n = jnp.sqrt(s)
        emb_ref[:] = h / jnp.maximum(n, 1e-12)

    rows = emb_ref[pl.ds(i * BM, BM), :]
    a = jax.lax.dot_general(
        rows, emb_ref[:],
        dimension_numbers=(((1,), (1,)), ((), ())),
        preferred_element_type=jnp.float32,
    )

    def body(_, carry):
        lo, hi, clo = carry
        mid = (lo + hi) * 0.5
        cnt = jnp.sum(jnp.where(a >= mid, 1.0, 0.0), axis=1, keepdims=True)
        ge = cnt >= K
        return (jnp.where(ge, mid, lo),
                jnp.where(ge, hi, mid),
                jnp.where(ge, cnt, clo))

    lo0 = jnp.full((BM, 1), -1.01, jnp.float32)
    hi0 = jnp.full((BM, 1), 1.01, jnp.float32)
    clo0 = jnp.full((BM, 1), float(N), jnp.float32)
    lo, _, clo = jax.lax.fori_loop(0, BISECT_ITERS, body, (lo0, hi0, clo0))
    candf = jnp.where(a >= lo, a, 2.0)
    t = jnp.min(candf, axis=1, keepdims=True)
    excess = clo - K
    for _ in range(N_STRIP):
        nxt = jnp.min(jnp.where(candf > t, candf, 2.0), axis=1, keepdims=True)
        t = jnp.where(excess >= 1.0, nxt, t)
        excess = excess - 1.0
    t = jnp.where(clo > K + N_STRIP, lo, t)
    t = jnp.maximum(t, 1.18e-38)
    out_ref[:] = jnp.where(a >= t, a, 0.0)


@jax.jit
def kernel(x, w0, w1):
    return pl.pallas_call(
        _fused_body,
        grid=(N // BM,),
        in_specs=[
            pl.BlockSpec((N, D), lambda i: (0, 0)),
            pl.BlockSpec((1, D), lambda i: (0, 0)),
            pl.BlockSpec((1, D), lambda i: (0, 0)),
        ],
        out_specs=pl.BlockSpec((BM, N), lambda i: (i, 0)),
        out_shape=jax.ShapeDtypeStruct((N, N), jnp.float32),
        scratch_shapes=[pltpu.VMEM((N, D), jnp.float32)],
    )(x, w0.reshape(1, D), w1.reshape(1, D))


# speedup vs baseline: 1.2818x; 1.0470x over previous
import jax
import jax.numpy as jnp
from jax.experimental import pallas as pl
from jax.experimental.pallas import tpu as pltpu

N = 4096
D = 512
K = 31
BM = 512
BISECT_ITERS = 13
N_STRIP = 4


def _fused_body(x_ref, w0_ref, w1_ref, out_ref, emb_ref):
    i = pl.program_id(0)

    @pl.when(i == 0)
    def _encode():
        h = x_ref[:] * w0_ref[:]
        h = jnp.maximum(h, 0.0)
        h = h * w1_ref[:]
        s = jnp.sum(h * h, axis=-1, keepdims=True)
        n = jnp.sqrt(s)
        emb_ref[:] = h / jnp.maximum(n, 1e-12)

    rows = emb_ref[pl.ds(i * BM, BM), :]
    a = jax.lax.dot_general(
        rows, emb_ref[:],
        dimension_numbers=(((1,), (1,)), ((), ())),
        preferred_element_type=jnp.float32,
    )

    def body(_, carry):
        lo, hi, clo = carry
        mid = (lo + hi) * 0.5
        cnt = jnp.sum(jnp.where(a >= mid, 1.0, 0.0), axis=1, keepdims=True)
        ge = cnt >= K
        return (jnp.where(ge, mid, lo),
                jnp.where(ge, hi, mid),
                jnp.where(ge, cnt, clo))

    lo0 = jnp.zeros((BM, 1), jnp.float32)
    hi0 = jnp.full((BM, 1), 1.01, jnp.float32)
    clo0 = jnp.full((BM, 1), float(N), jnp.float32)
    lo, _, clo = jax.lax.fori_loop(0, BISECT_ITERS, body, (lo0, hi0, clo0))
    candf = jnp.where(a >= lo, a, 2.0)
    t = jnp.min(candf, axis=1, keepdims=True)
    excess = clo - K
    for _ in range(N_STRIP):
        nxt = jnp.min(jnp.where(candf > t, candf, 2.0), axis=1, keepdims=True)
        t = jnp.where(excess >= 1.0, nxt, t)
        excess = excess - 1.0
    t = jnp.where(clo > K + N_STRIP, lo, t)
    t = jnp.maximum(t, 1.18e-38)
    out_ref[:] = jnp.where(a >= t, a, 0.0)


@jax.jit
def kernel(x, w0, w1):
    return pl.pallas_call(
        _fused_body,
        grid=(N // BM,),
        in_specs=[
            pl.BlockSpec((N, D), lambda i: (0, 0)),
            pl.BlockSpec((1, D), lambda i: (0, 0)),
            pl.BlockSpec((1, D), lambda i: (0, 0)),
        ],
        out_specs=pl.BlockSpec((BM, N), lambda i: (i, 0)),
        out_shape=jax.ShapeDtypeStruct((N, N), jnp.float32),
        scratch_shapes=[pltpu.VMEM((N, D), jnp.float32)],
    )(x, w0.reshape(1, D), w1.reshape(1, D))
